# Initial kernel scaffold; baseline (speedup 1.0000x reference)
import jax
import jax.numpy as jnp
from jax.experimental import pallas as pl


def kernel(coordinate, W0, b0, g0, be0, W1, b1, g1, be1, W2, b2, g2, be2, W3, b3):
    raise NotImplementedError("write your pallas kernel here")


# speedup vs baseline: 5.5572x; 5.5572x over previous
import functools

import jax
import jax.numpy as jnp
from jax import lax
from jax.experimental import pallas as pl
from jax.experimental.pallas import tpu as pltpu
from jax.experimental.pallas import tpu_sc as plsc

NLOC = 32
RADIUS2 = 1.0
EPS = 1e-5


def _topk_body(q_ref, out_ref, *, rs, m, k):
    x = q_ref[0]
    iota = lax.broadcasted_iota(jnp.int32, (rs, m), 1)
    kiota = lax.broadcasted_iota(jnp.int32, (1, k), 1)
    acc = jnp.zeros((rs, k), jnp.int32)

    def body(j, carry):
        x, acc = carry
        mx = jnp.max(x, axis=1, keepdims=True)
        cand = jnp.where(x >= mx, iota, m)
        idx = jnp.min(cand, axis=1, keepdims=True)
        acc = jnp.where(kiota == j, idx, acc)
        x = jnp.where(cand == idx, -1.0, x)
        return x, acc

    _, acc = lax.fori_loop(0, k, body, (x, acc))
    out_ref[0] = acc


def _topk(qs, rs, k):
    b_dim, s_dim, m = qs.shape
    grid = (b_dim, s_dim // rs)
    return pl.pallas_call(
        functools.partial(_topk_body, rs=rs, m=m, k=k),
        grid=grid,
        in_specs=[pl.BlockSpec((1, rs, m), lambda b, si: (b, si, 0))],
        out_specs=pl.BlockSpec((1, rs, k), lambda b, si: (b, si, 0)),
        out_shape=jax.ShapeDtypeStruct((b_dim, s_dim, k), jnp.int32),
    )(qs)


def _mask_sc(coord_flat, grouped_flat, sampled_exp_flat, b_dim, m, s_dim):
    info = plsc.get_sparse_core_info()
    nc, ns = info.num_cores, info.num_subcores
    nw = nc * ns
    rows = b_dim * s_dim
    rpw = rows // nw
    wpb = nw // b_dim
    mesh = plsc.VectorSubcoreMesh(core_axis_name="c", subcore_axis_name="s")

    @functools.partial(
        pl.kernel,
        out_type=jax.ShapeDtypeStruct((rows * NLOC,), jnp.int32),
        mesh=mesh,
        compiler_params=pltpu.CompilerParams(needs_layout_passes=False),
        scratch_types=[
            pltpu.VMEM((m * 3,), jnp.float32),
            pltpu.VMEM((rpw * NLOC,), jnp.int32),
            pltpu.VMEM((rpw * NLOC,), jnp.int32),
            pltpu.VMEM((rpw * NLOC,), jnp.int32),
        ],
    )
    def run(coord_hbm, grouped_hbm, sampled_hbm, out_hbm,
            coord_v, gi_v, se_v, out_v):
        wid = lax.axis_index("s") * nc + lax.axis_index("c")
        b = wid // wpb
        row0 = wid * rpw
        pltpu.sync_copy(coord_hbm.at[pl.ds(b * m * 3, m * 3)], coord_v)
        pltpu.sync_copy(grouped_hbm.at[pl.ds(row0 * NLOC, rpw * NLOC)], gi_v)
        pltpu.sync_copy(sampled_hbm.at[pl.ds(row0 * NLOC, rpw * NLOC)], se_v)

        def chunk_body(t, _):
            off = t * 16
            sv = se_v[pl.ds(off, 16)]
            sx = plsc.load_gather(coord_v, [sv * 3])
            sy = plsc.load_gather(coord_v, [sv * 3 + 1])
            sz = plsc.load_gather(coord_v, [sv * 3 + 2])
            gi = gi_v[pl.ds(off, 16)]
            gx = plsc.load_gather(coord_v, [gi * 3])
            gy = plsc.load_gather(coord_v, [gi * 3 + 1])
            gz = plsc.load_gather(coord_v, [gi * 3 + 2])
            dx = gx - sx
            dy = gy - sy
            dz = gz - sz
            d2 = dx * dx + dy * dy + dz * dz
            out_v[pl.ds(off, 16)] = jnp.where(d2 > RADIUS2, sv, gi)
            return 0

        lax.fori_loop(0, rpw * NLOC // 16, chunk_body, 0)
        pltpu.sync_copy(out_v, out_hbm.at[pl.ds(row0 * NLOC, rpw * NLOC)])

    return run(coord_flat, grouped_flat, sampled_exp_flat)


def _conv1d(x, W, b):
    return jnp.einsum('oi,bim->bom', W, x) + b[None, :, None]


def _bn(x, gamma, beta):
    mean = jnp.mean(x, axis=(0, 2), keepdims=True)
    var = jnp.mean((x - mean) ** 2, axis=(0, 2), keepdims=True)
    xn = (x - mean) / jnp.sqrt(var + EPS)
    return gamma[None, :, None] * xn + beta[None, :, None]


def kernel(coordinate, W0, b0, g0, be0, W1, b1, g1, be1, W2, b2, g2, be2,
           W3, b3):
    b_dim, m, _ = coordinate.shape
    s_dim = W3.shape[0]
    rs = min(128, s_dim)

    x = jnp.transpose(coordinate, (0, 2, 1))
    x = jax.nn.relu(_bn(_conv1d(x, W0, b0), g0, be0))
    x = jax.nn.relu(_bn(_conv1d(x, W1, b1), g1, be1))
    x = jax.nn.relu(_bn(_conv1d(x, W2, b2), g2, be2))
    x = _conv1d(x, W3, b3)
    qs = jax.nn.softmax(x, axis=1)

    grouped_raw = _topk(qs, rs, NLOC)
    sampled = grouped_raw[:, :, 0]

    sampled_exp = jnp.repeat(sampled[:, :, None], NLOC, axis=2)
    masked = _mask_sc(coordinate.reshape(-1),
                      grouped_raw.reshape(-1),
                      sampled_exp.reshape(-1),
                      b_dim, m, s_dim)
    grouped = masked.reshape(b_dim, s_dim, NLOC)
    return sampled, grouped
